# Initial kernel scaffold; baseline (speedup 1.0000x reference)
import jax
import jax.numpy as jnp
from jax.experimental import pallas as pl


def kernel(x, spatial_edge_index, spatial_edge_attr, dom_edge_index, dom_edge_attr, mask, Wt, bt, Wp1, bp1, Wp2, bp2, Wd1, bd1, Wd2, bd2):
    raise NotImplementedError("write your pallas kernel here")


# speedup vs baseline: 15.0947x; 15.0947x over previous
import functools

import jax
import jax.numpy as jnp
from jax import lax
from jax.experimental import pallas as pl
from jax.experimental.pallas import tpu as pltpu
from jax.experimental.pallas import tpu_sc as plsc

N = 10000
E = 320000
NP = 10240
SL = NP // 16
ET = E // 16
EWK = 2 * E // 32
CH = 80
NCH = EWK // CH
LANES = 16


def _node_mm_body(x_ref, w_ref, o_ref):
    o_ref[...] = jnp.dot(x_ref[...], w_ref[...],
                         preferred_element_type=jnp.float32)


def _node_matmul(x, w):
    blk = 1000
    return pl.pallas_call(
        _node_mm_body,
        grid=(N // blk,),
        in_specs=[pl.BlockSpec((blk, 128), lambda i: (i, 0)),
                  pl.BlockSpec((128, 32), lambda i: (0, 0))],
        out_specs=pl.BlockSpec((blk, 32), lambda i: (i, 0)),
        out_shape=jax.ShapeDtypeStruct((N, 32), jnp.float32),
    )(x, w)


def _gather_sum_body(xa_hbm, xb_hbm, src_hbm, dst_hbm, out_hbm,
                     sidx, didx, abuf, bbuf, obuf, sema, semb):
    c = lax.axis_index("c")
    s = lax.axis_index("s")
    wid = c * 16 + s
    base = wid * EWK
    pltpu.sync_copy(src_hbm.at[pl.ds(base, EWK)], sidx)
    pltpu.sync_copy(dst_hbm.at[pl.ds(base, EWK)], didx)

    def fire(g, slot):
        pltpu.async_copy(xa_hbm.at[sidx.at[pl.ds(g * CH, CH)]],
                         abuf.at[slot], sema)
        pltpu.async_copy(xb_hbm.at[didx.at[pl.ds(g * CH, CH)]],
                         bbuf.at[slot], semb)

    fire(0, 0)
    fire(1, 1)

    def outer(i, carry):
        for b in range(2):
            g = i * 2 + b
            pltpu.make_async_copy(xa_hbm.at[sidx.at[pl.ds(0, CH)]],
                                  abuf.at[b], sema).wait()
            pltpu.make_async_copy(xb_hbm.at[didx.at[pl.ds(0, CH)]],
                                  bbuf.at[b], semb).wait()
            for r in range(CH):
                obuf[b, r] = abuf[b, r] + bbuf[b, r]
            pltpu.sync_copy(obuf.at[b],
                            out_hbm.at[pl.ds(base + g * CH, CH)])

            @pl.when(g + 2 < NCH)
            def _():
                fire(g + 2, b)
        return carry

    lax.fori_loop(0, NCH // 2, outer, 0)


def _gather_sum(xa, xb, src, dst):
    mesh = plsc.VectorSubcoreMesh(core_axis_name="c", subcore_axis_name="s")
    f = pl.kernel(
        _gather_sum_body,
        out_type=jax.ShapeDtypeStruct((2 * E, 16), jnp.float32),
        mesh=mesh,
        compiler_params=pltpu.CompilerParams(use_tc_tiling_on_sc=False, needs_layout_passes=False),
        scratch_types=[
            pltpu.VMEM((EWK,), jnp.int32),
            pltpu.VMEM((EWK,), jnp.int32),
            pltpu.VMEM((2, CH, 16), jnp.float32),
            pltpu.VMEM((2, CH, 16), jnp.float32),
            pltpu.VMEM((2, CH, 16), jnp.float32),
            pltpu.SemaphoreType.DMA,
            pltpu.SemaphoreType.DMA,
        ],
    )
    return f(xa, xb, src, dst)


def _mlp_body(t_ref, a_ref, bt_ref, w1a_ref, w1b_ref, b1_ref, w2_ref, b2_ref,
              o_ref):
    trans = jax.nn.relu(t_ref[...] + bt_ref[...])
    h = jax.nn.relu(a_ref[...] @ w1a_ref[0] + trans @ w1b_ref[0]
                    + b1_ref[0])
    o_ref[...] = jax.nn.sigmoid(h @ w2_ref[0] + b2_ref[0])


def _edge_mlp(tsum, attr2, btr, w1a2, w1b2, b12, w22, b22):
    blk = 2000
    nblk = E // blk
    return pl.pallas_call(
        _mlp_body,
        grid=(2, nblk),
        in_specs=[
            pl.BlockSpec((blk, 16), lambda b, i: (b * nblk + i, 0)),
            pl.BlockSpec((blk, 4), lambda b, i: (b * nblk + i, 0)),
            pl.BlockSpec((1, 16), lambda b, i: (0, 0)),
            pl.BlockSpec((1, 4, 32), lambda b, i: (b, 0, 0)),
            pl.BlockSpec((1, 16, 32), lambda b, i: (b, 0, 0)),
            pl.BlockSpec((1, 1, 32), lambda b, i: (b, 0, 0)),
            pl.BlockSpec((1, 32, 1), lambda b, i: (b, 0, 0)),
            pl.BlockSpec((1, 1, 1), lambda b, i: (b, 0, 0)),
        ],
        out_specs=pl.BlockSpec((blk, 1), lambda b, i: (b * nblk + i, 0)),
        out_shape=jax.ShapeDtypeStruct((2 * E, 1), jnp.float32),
    )(tsum, attr2, btr, w1a2, w1b2, b12, w22, b22)


def _prop_body(src_hbm, dst_hbm, ew_hbm, m0_hbm, out_hbm,
               isrc, idst, wv, m_in, m_out, mrg, msl, sh_all, sh_merged):
    c = lax.axis_index("c")
    s = lax.axis_index("s")
    base = s * ET
    pltpu.sync_copy(src_hbm.at[c, pl.ds(base, ET)], isrc)
    pltpu.sync_copy(dst_hbm.at[c, pl.ds(base, ET)], idst)
    pltpu.sync_copy(ew_hbm.at[c, pl.ds(base, ET)], wv)
    pltpu.sync_copy(m0_hbm, m_in)

    def copy_m(i, carry):
        k = i * LANES
        m_out[pl.ds(k, LANES)] = m_in[pl.ds(k, LANES)]
        return carry

    def edge(i, carry):
        k = i * LANES
        si = isrc[pl.ds(k, LANES)]
        di = idst[pl.ds(k, LANES)]
        v = wv[pl.ds(k, LANES)] * plsc.load_gather(m_in, [si])

        def wbody(act):
            cur = plsc.load_gather(m_out, [di])
            plsc.store_scatter(m_out, [di], jnp.maximum(cur, v), mask=act)
            chk = plsc.load_gather(m_out, [di])
            return jnp.logical_and(act, chk < v)

        lax.while_loop(lambda a: jnp.any(a), wbody,
                       jnp.ones((LANES,), jnp.bool_))
        return carry

    def reduce_slice(i, carry):
        k = i * LANES
        acc = mrg[0, pl.ds(k, LANES)]
        for t in range(1, 16):
            acc = jnp.maximum(acc, mrg[t, pl.ds(k, LANES)])
        msl[pl.ds(k, LANES)] = acc
        return carry

    for rnd in range(3):
        lax.fori_loop(0, NP // LANES, copy_m, 0)
        lax.fori_loop(0, ET // LANES, edge, 0)
        pltpu.sync_copy(m_out, sh_all.at[s])
        plsc.subcore_barrier()
        for t in range(16):
            pltpu.sync_copy(sh_all.at[t, pl.ds(s * SL, SL)], mrg.at[t])
        lax.fori_loop(0, SL // LANES, reduce_slice, 0)
        if rnd < 2:
            pltpu.sync_copy(msl, sh_merged.at[pl.ds(s * SL, SL)])
            plsc.subcore_barrier()
            pltpu.sync_copy(sh_merged, m_in)
        else:
            pltpu.sync_copy(msl, out_hbm.at[c, pl.ds(s * SL, SL)])


def _propagate(src2, dst2, ew2, m0p):
    mesh = plsc.VectorSubcoreMesh(core_axis_name="c", subcore_axis_name="s")
    f = pl.kernel(
        _prop_body,
        out_type=jax.ShapeDtypeStruct((2, NP), jnp.float32),
        mesh=mesh,
        compiler_params=pltpu.CompilerParams(use_tc_tiling_on_sc=False, needs_layout_passes=False),
        scratch_types=[
            pltpu.VMEM((ET,), jnp.int32),
            pltpu.VMEM((ET,), jnp.int32),
            pltpu.VMEM((ET,), jnp.float32),
            pltpu.VMEM((NP,), jnp.float32),
            pltpu.VMEM((NP,), jnp.float32),
            pltpu.VMEM((16, SL), jnp.float32),
            pltpu.VMEM((SL,), jnp.float32),
            pltpu.VMEM_SHARED((16, NP), jnp.float32),
            pltpu.VMEM_SHARED((NP,), jnp.float32),
        ],
    )
    return f(src2, dst2, ew2, m0p)


def kernel(x, spatial_edge_index, spatial_edge_attr, dom_edge_index,
           dom_edge_attr, mask, Wt, bt, Wp1, bp1, Wp2, bp2, Wd1, bd1,
           Wd2, bd2):
    w = jnp.concatenate([Wt[:128], Wt[128:]], axis=1)
    xw = _node_matmul(x, w)
    xa = jnp.pad(xw[:, :16], ((0, NP - N), (0, 0)))
    xb = jnp.pad(xw[:, 16:], ((0, NP - N), (0, 0)))

    src = jnp.concatenate([spatial_edge_index[0], dom_edge_index[0]])
    dst = jnp.concatenate([spatial_edge_index[1], dom_edge_index[1]])

    tsum = _gather_sum(xa, xb, src, dst)

    attr2 = jnp.concatenate(
        [spatial_edge_attr,
         jnp.pad(dom_edge_attr, ((0, 0), (0, 3)))], axis=0)
    w1a2 = jnp.stack([Wp1[:4], jnp.pad(Wd1[:1], ((0, 3), (0, 0)))])
    w1b2 = jnp.stack([Wp1[4:], Wd1[1:]])
    b12 = jnp.stack([bp1.reshape(1, 32), bd1.reshape(1, 32)])
    w22 = jnp.stack([Wp2, Wd2])
    b22 = jnp.stack([bp2.reshape(1, 1), bd2.reshape(1, 1)])
    ew = _edge_mlp(tsum, attr2, bt.reshape(1, 16), w1a2, w1b2, b12, w22, b22)
    ew2 = ew.reshape(2, E)

    src2 = src.reshape(2, E)
    dst2 = dst.reshape(2, E)
    m0p = jnp.pad(mask, (0, NP - N))
    mout = _propagate(src2, dst2, ew2, m0p)

    return jnp.maximum(mout[0, :N], mout[1, :N])
